# baseline (speedup 1.0000x reference)
import functools
import math

import jax
import jax.numpy as jnp
import numpy as np
from jax import lax
from jax.experimental import pallas as pl
from jax.experimental.pallas import tpu as pltpu
from jax.experimental.pallas import tpu_sc as plsc

_WINDOW_SIZE = 10


def _prep_body(ts_ref, emb_ref, t_ref, tt_ref, *, num_bins, s, d):
    ts = ts_ref[...]
    rt = (ts - ts[:, 0:1]).astype(jnp.float32)
    mx = jnp.max(rt)
    st = jnp.clip(rt / mx, 0.0, float(num_bins - 1)).astype(jnp.int32)
    t_ref[...] = jnp.sum((st == 0).astype(jnp.int32), axis=1, keepdims=True)
    j = lax.broadcasted_iota(jnp.int32, (num_bins, num_bins), 1).astype(jnp.float32)
    k = lax.broadcasted_iota(jnp.int32, (num_bins, num_bins), 0).astype(jnp.float32)
    w = jnp.cos((math.pi / _WINDOW_SIZE) * (j - k)) ** 2
    wsum = jnp.sum(w, axis=1, keepdims=True)
    l = s * d
    li2 = lax.broadcasted_iota(jnp.int32, (d, l), 1)
    di = lax.broadcasted_iota(jnp.int32, (d, l), 0)
    c_mat = (li2 % d == di).astype(jnp.float32)
    r0l = jnp.dot(emb_ref[0:1, :] * wsum[0, 0], c_mat,
                  preferred_element_type=jnp.float32)
    r1l = jnp.dot(emb_ref[1:2, :] * wsum[1, 0], c_mat,
                  preferred_element_type=jnp.float32)
    ti = lax.broadcasted_iota(jnp.int32, (s + 1, l), 0)
    si = lax.broadcasted_iota(jnp.int32, (s + 1, l), 1) // d
    tt_ref[...] = jnp.where(si < ti, r0l, r1l)


def _make_sc_lookup(n_b, row_w):
    info = plsc.get_sparse_core_info()
    nc, ns = info.num_cores, info.num_subcores
    nw = nc * ns
    rows_per_w = n_b // nw
    mesh = plsc.VectorSubcoreMesh(core_axis_name="c", subcore_axis_name="s")

    n_chunks = 4
    chunk = rows_per_w // n_chunks

    @functools.partial(
        pl.kernel,
        mesh=mesh,
        out_type=jax.ShapeDtypeStruct((n_b, row_w), jnp.float32),
        scratch_types=[
            pltpu.VMEM((n_chunks, chunk), jnp.int32),
            pltpu.VMEM((rows_per_w, row_w), jnp.float32),
            [pltpu.SemaphoreType.DMA] * n_chunks,
            pltpu.SemaphoreType.DMA,
        ],
        compiler_params=pltpu.CompilerParams(use_tc_tiling_on_sc=False),
    )
    def sc_lookup(tt_hbm, idx_hbm, out_hbm, idx_v, rows_v, gsems, osem):
        wid = lax.axis_index("s") * nc + lax.axis_index("c")
        base = wid * rows_per_w
        pltpu.sync_copy(
            idx_hbm.at[pl.ds(wid * n_chunks, n_chunks)], idx_v
        )
        gathers = [
            pltpu.async_copy(
                tt_hbm.at[idx_v.at[j]],
                rows_v.at[pl.ds(j * chunk, chunk)],
                gsems[j],
            )
            for j in range(n_chunks)
        ]
        outs = []
        for j in range(n_chunks):
            gathers[j].wait()
            outs.append(
                pltpu.async_copy(
                    rows_v.at[pl.ds(j * chunk, chunk)],
                    out_hbm.at[pl.ds(base + j * chunk, chunk)],
                    osem,
                )
            )
        for cp in outs:
            cp.wait()

    return sc_lookup


def kernel(timestamps, time_embedding):
    b, s = timestamps.shape
    num_bins, d = time_embedding.shape
    prep = pl.pallas_call(
        functools.partial(_prep_body, num_bins=num_bins, s=s, d=d),
        out_shape=(
            jax.ShapeDtypeStruct((b, 1), jnp.int32),
            jax.ShapeDtypeStruct((s + 1, s * d), jnp.float32),
        ),
    )
    t, tt = prep(timestamps, time_embedding)
    out = _make_sc_lookup(b, s * d)(tt, t.reshape(b // 8, 8))
    return out.reshape(b, s, d)


# speedup vs baseline: 1.3989x; 1.3989x over previous
import functools
import math

import jax
import jax.numpy as jnp
import numpy as np
from jax import lax
from jax.experimental import pallas as pl
from jax.experimental.pallas import tpu as pltpu
from jax.experimental.pallas import tpu_sc as plsc

_WINDOW_SIZE = 10


def _prep_body(ts_ref, emb_ref, t_ref, strip_ref, *, num_bins, s):
    ts = ts_ref[...]
    rt = (ts - ts[:, 0:1]).astype(jnp.float32)
    mx = jnp.max(rt)
    st = jnp.clip(rt / mx, 0.0, float(num_bins - 1)).astype(jnp.int32)
    t = jnp.sum((st == 0).astype(jnp.int32), axis=1, keepdims=True)
    d = emb_ref.shape[1]
    t_ref[...] = (s - t) * d
    j = lax.broadcasted_iota(jnp.int32, (num_bins, num_bins), 1).astype(jnp.float32)
    k = lax.broadcasted_iota(jnp.int32, (num_bins, num_bins), 0).astype(jnp.float32)
    w = jnp.cos((math.pi / _WINDOW_SIZE) * (j - k)) ** 2
    wsum = jnp.sum(w, axis=1, keepdims=True)
    r0 = emb_ref[0:1, :] * wsum[0, 0]
    r1 = emb_ref[1:2, :] * wsum[1, 0]
    ri = lax.broadcasted_iota(jnp.int32, (2 * s, 1), 0)
    strip_ref[...] = jnp.where(ri < s, r0, r1)


def _make_sc_windows(n_b, s, d):
    info = plsc.get_sparse_core_info()
    nc, ns = info.num_cores, info.num_subcores
    nw = nc * ns
    rows_per_w = n_b // nw
    row_w = s * d
    strip_n = 2 * s * d

    mesh = plsc.VectorSubcoreMesh(core_axis_name="c", subcore_axis_name="s")

    @functools.partial(
        pl.kernel,
        mesh=mesh,
        out_type=jax.ShapeDtypeStruct((n_b * row_w,), jnp.float32),
        scratch_types=[
            pltpu.VMEM((rows_per_w,), jnp.int32),
            pltpu.VMEM((strip_n,), jnp.float32),
            pltpu.SemaphoreType.DMA,
            pltpu.SemaphoreType.DMA,
        ],
        compiler_params=pltpu.CompilerParams(use_tc_tiling_on_sc=False),
    )
    def sc_windows(strip_hbm, off_hbm, out_hbm, off_v, strip_v, ssem, osem):
        wid = lax.axis_index("s") * nc + lax.axis_index("c")
        base = wid * rows_per_w
        cp_s = pltpu.async_copy(strip_hbm, strip_v, ssem)
        pltpu.sync_copy(off_hbm.at[pl.ds(base, rows_per_w)], off_v)
        cp_s.wait()
        offs = [off_v[pl.ds(g * 16, 16)] for g in range(rows_per_w // 16)]
        outs = []
        for i in range(rows_per_w):
            off = pl.multiple_of(offs[i // 16][i % 16], d)
            outs.append(
                pltpu.async_copy(
                    strip_v.at[pl.ds(off, row_w)],
                    out_hbm.at[pl.ds(pl.multiple_of((base + i) * row_w, row_w), row_w)],
                    osem,
                )
            )
        for cp in outs:
            cp.wait()

    return sc_windows


def kernel(timestamps, time_embedding):
    b, s = timestamps.shape
    num_bins, d = time_embedding.shape
    prep = pl.pallas_call(
        functools.partial(_prep_body, num_bins=num_bins, s=s),
        out_shape=(
            jax.ShapeDtypeStruct((b, 1), jnp.int32),
            jax.ShapeDtypeStruct((2 * s, d), jnp.float32),
        ),
    )
    off, strip = prep(timestamps, time_embedding)
    out = _make_sc_windows(b, s, d)(strip.reshape(2 * s * d), off.reshape(b))
    return out.reshape(b, s, d)


# speedup vs baseline: 1.4002x; 1.0009x over previous
import functools
import math

import jax
import jax.numpy as jnp
from jax import lax
from jax.experimental import pallas as pl
from jax.experimental.pallas import tpu as pltpu
from jax.experimental.pallas import tpu_sc as plsc

_WINDOW_SIZE = 10


def _prep_body(ts_ref, emb_ref, t_ref, strip_ref, *, num_bins, s):
    ts = ts_ref[...]
    rt = (ts - ts[:, 0:1]).astype(jnp.float32)
    mx = jnp.max(rt)
    st = jnp.clip(rt / mx, 0.0, float(num_bins - 1)).astype(jnp.int32)
    t = jnp.sum((st == 0).astype(jnp.int32), axis=1, keepdims=True)
    d = emb_ref.shape[1]
    t_ref[...] = (s - t) * d
    j = lax.broadcasted_iota(jnp.int32, (num_bins, num_bins), 1).astype(jnp.float32)
    k = lax.broadcasted_iota(jnp.int32, (num_bins, num_bins), 0).astype(jnp.float32)
    w = jnp.cos((math.pi / _WINDOW_SIZE) * (j - k)) ** 2
    wsum = jnp.sum(w, axis=1, keepdims=True)
    r0 = emb_ref[0:1, :] * wsum[0, 0]
    r1 = emb_ref[1:2, :] * wsum[1, 0]
    ri = lax.broadcasted_iota(jnp.int32, (2 * s, 1), 0)
    strip_ref[...] = jnp.where(ri < s, r0, r1)


def _make_sc_windows(n_b, s, d):
    info = plsc.get_sparse_core_info()
    nc, ns = info.num_cores, info.num_subcores
    nw = nc * ns
    rows_per_w = n_b // nw
    row_w = s * d
    strip_n = 2 * s * d

    mesh = plsc.VectorSubcoreMesh(core_axis_name="c", subcore_axis_name="s")

    @functools.partial(
        pl.kernel,
        mesh=mesh,
        out_type=jax.ShapeDtypeStruct((n_b * row_w,), jnp.float32),
        scratch_types=[
            pltpu.VMEM((rows_per_w,), jnp.int32),
            pltpu.VMEM((strip_n,), jnp.float32),
            pltpu.SemaphoreType.DMA,
            pltpu.SemaphoreType.DMA,
        ],
        compiler_params=pltpu.CompilerParams(use_tc_tiling_on_sc=False),
    )
    def sc_windows(strip_hbm, off_hbm, out_hbm, off_v, strip_v, ssem, osem):
        wid = lax.axis_index("s") * nc + lax.axis_index("c")
        base = wid * rows_per_w
        cp_s = pltpu.async_copy(strip_hbm, strip_v, ssem)
        pltpu.sync_copy(off_hbm.at[pl.ds(base, rows_per_w)], off_v)
        cp_s.wait()
        offs = [off_v[pl.ds(g * 16, 16)] for g in range(rows_per_w // 16)]
        outs = []
        for i in range(rows_per_w):
            off = pl.multiple_of(offs[i // 16][i % 16], d)
            outs.append(
                pltpu.async_copy(
                    strip_v.at[pl.ds(off, row_w)],
                    out_hbm.at[pl.ds(pl.multiple_of((base + i) * row_w, row_w), row_w)],
                    osem,
                )
            )
        for cp in outs:
            cp.wait()

    return sc_windows


def kernel(timestamps, time_embedding):
    b, s = timestamps.shape
    num_bins, d = time_embedding.shape
    prep = pl.pallas_call(
        functools.partial(_prep_body, num_bins=num_bins, s=s),
        out_shape=(
            jax.ShapeDtypeStruct((b, 1), jnp.int32),
            jax.ShapeDtypeStruct((2 * s, d), jnp.float32),
        ),
    )
    off, strip = prep(timestamps, time_embedding)
    out = _make_sc_windows(b, s, d)(strip.reshape(2 * s * d), off.reshape(b))
    return out.reshape(b, s, d)
